# baseline (speedup 1.0000x reference)
import functools

import numpy as np

import jax
import jax.numpy as jnp
from jax import lax
from jax.experimental import pallas as pl
from jax.experimental.pallas import tpu as pltpu
from jax.experimental.pallas import tpu_sc as plsc

N = 10000
EM = 160000
EW = 32000
LAT = 128
LN_EPS = 1e-5
NC = 2
NS = 16
NW = NC * NS
CH = 128
NP_ = 10240
SPT = NP_ // NS

_f32 = jnp.float32


def _rb(blk, d):
    return pl.BlockSpec((blk, d), lambda i: (i, 0))


def _full(shape):
    nd = len(shape)
    return pl.BlockSpec(shape, lambda i: (0,) * nd)


def _ln(h, g, b):
    mu = jnp.mean(h, axis=-1, keepdims=True)
    d = h - mu
    var = jnp.mean(d * d, axis=-1, keepdims=True)
    return d * lax.rsqrt(var + LN_EPS) * g + b


def _dot(a, b):
    return jnp.dot(a, b, preferred_element_type=_f32)


def _enc_body(x_ref, w1, w2, w3, w4, b1, b2, b3, b4, g, be, o_ref):
    h = jnp.maximum(_dot(x_ref[...], w1[...]) + b1[...], 0.0)
    h = jnp.maximum(_dot(h, w2[...]) + b2[...], 0.0)
    h = jnp.maximum(_dot(h, w3[...]) + b3[...], 0.0)
    h = _dot(h, w4[...]) + b4[...]
    o_ref[...] = _ln(h, g[...], be[...])


def _enc_call(x, p, blk):
    rows, din = x.shape
    Ws, bs = p["W"], p["b"]
    args = [x, Ws[0], Ws[1], Ws[2], Ws[3]] + [b.reshape(1, -1) for b in bs] \
        + [p["g"].reshape(1, -1), p["be"].reshape(1, -1)]
    specs = [_rb(blk, din)] + [_full(a.shape) for a in args[1:]]
    return pl.pallas_call(
        _enc_body,
        grid=(rows // blk,),
        in_specs=specs,
        out_specs=_rb(blk, LAT),
        out_shape=jax.ShapeDtypeStruct((rows, LAT), _f32),
    )(*args)


def _proj_body(x_ref, wam, wbm, b1m, waw, wbw, b1w, psm, prm, psw, prw):
    x = x_ref[...]
    psm[...] = _dot(x, wam[...])
    prm[...] = _dot(x, wbm[...]) + b1m[...]
    psw[...] = _dot(x, waw[...])
    prw[...] = _dot(x, wbw[...]) + b1w[...]


def _proj_call(nl, wam, wbm, b1m, waw, wbw, b1w, blk=5000):
    args = [nl, wam, wbm, b1m.reshape(1, -1), waw, wbw, b1w.reshape(1, -1)]
    specs = [_rb(blk, LAT)] + [_full(a.shape) for a in args[1:]]
    sh = jax.ShapeDtypeStruct((N, LAT), _f32)
    return pl.pallas_call(
        _proj_body,
        grid=(N // blk,),
        in_specs=specs,
        out_specs=[_rb(blk, LAT)] * 4,
        out_shape=[sh, sh, sh, sh],
    )(*args)


def _bdot(a, b):
    return jnp.dot(a.astype(jnp.bfloat16), b.astype(jnp.bfloat16),
                   preferred_element_type=_f32)


def _edge_body(g_ref, lat_ref, w1c, w2, w3, w4, b2, b3, b4, gm, be,
               new_ref, lat_o_ref):
    x = lat_ref[...]
    h = jnp.maximum(g_ref[...] + _bdot(x, w1c[...]), 0.0)
    h = jnp.maximum(_bdot(h, w2[...]) + b2[...], 0.0)
    h = jnp.maximum(_bdot(h, w3[...]) + b3[...], 0.0)
    h = _bdot(h, w4[...]) + b4[...]
    nm = _ln(h, gm[...], be[...])
    new_ref[...] = nm
    lat_o_ref[...] = x + nm


def _edge_call(G, lat, p, w1c, blk=6400):
    rows = lat.shape[0]
    Ws, bs = p["W"], p["b"]
    args = [G, lat, w1c, Ws[1], Ws[2], Ws[3],
            bs[1].reshape(1, -1), bs[2].reshape(1, -1), bs[3].reshape(1, -1),
            p["g"].reshape(1, -1), p["be"].reshape(1, -1)]
    specs = [_rb(blk, LAT), _rb(blk, LAT)] + [_full(a.shape) for a in args[2:]]
    sh = jax.ShapeDtypeStruct((rows, LAT), _f32)
    return pl.pallas_call(
        _edge_body,
        grid=(rows // blk,),
        in_specs=specs,
        out_specs=[_rb(blk, LAT), _rb(blk, LAT)],
        out_shape=[sh, sh],
    )(*args)


def _node_body(nl_ref, am0, am1, aw0, aw1, v1a, v1b, v1c, c1, v2, v3, v4,
               c2, c3, c4, gm, be, o_ref):
    x = nl_ref[...]
    h = (_dot(x, v1a[...]) + _dot(am0[...] + am1[...], v1b[...])
         + _dot(aw0[...] + aw1[...], v1c[...]) + c1[...])
    h = jnp.maximum(h, 0.0)
    h = jnp.maximum(_dot(h, v2[...]) + c2[...], 0.0)
    h = jnp.maximum(_dot(h, v3[...]) + c3[...], 0.0)
    h = _dot(h, v4[...]) + c4[...]
    o_ref[...] = x + _ln(h, gm[...], be[...])


def _node_call(nl, am0, am1, aw0, aw1, p, blk=5000):
    Ws, bs = p["W"], p["b"]
    V1 = Ws[0]
    args = [nl, am0, am1, aw0, aw1,
            V1[:LAT], V1[LAT:2 * LAT], V1[2 * LAT:], bs[0].reshape(1, -1),
            Ws[1], Ws[2], Ws[3],
            bs[1].reshape(1, -1), bs[2].reshape(1, -1), bs[3].reshape(1, -1),
            p["g"].reshape(1, -1), p["be"].reshape(1, -1)]
    specs = [_rb(blk, LAT)] * 5 + [_full(a.shape) for a in args[5:]]
    return pl.pallas_call(
        _node_body,
        grid=(N // blk,),
        in_specs=specs,
        out_specs=_rb(blk, LAT),
        out_shape=jax.ShapeDtypeStruct((N, LAT), _f32),
    )(*args)


def _chunking(E, nworkers):
    tc = E // CH
    q, r = divmod(tc, nworkers)
    nch_max = q + (1 if r else 0)
    nch_pad = -(-nch_max // 8) * 8
    return tc, q, r, nch_max, nch_pad

def _sc_mesh():
    return plsc.VectorSubcoreMesh(
        core_axis_name="c", subcore_axis_name="s", num_cores=NC,
        num_subcores=NS)


def _gather_pipe(ps, pr, isv, irv, out, q, r, tc_, nch_pad, wid,
                 rs, rr, sg, sw):
    nch = q + jnp.where(wid < r, 1, 0)
    start = wid * q + jnp.minimum(wid, r)
    rsb, rrb = rs[0], rr[0]

    def body(c, carry):
        d1 = pltpu.async_copy(ps.at[isv.at[c]], rsb, sg[0])
        d2 = pltpu.async_copy(pr.at[irv.at[c]], rrb, sg[1])
        d1.wait()
        d2.wait()

        def addb(rw, c2):
            for k8 in range(LAT // 16):
                s_ = pl.ds(k8 * 16, 16)
                rsb[rw, s_] = rsb[rw, s_] + rrb[rw, s_]
            return c2

        lax.fori_loop(0, CH, addb, 0)
        pltpu.sync_copy(rsb, out.at[pl.ds((start + c) * CH, CH)])
        return carry

    lax.fori_loop(0, nch, body, 0)


@functools.cache
def _make_gather():
    tcm, qm, rm, _, npm = _chunking(EM, NW)
    tcw, qw, rw, _, npw = _chunking(EW, NW)

    @functools.partial(
        pl.kernel, mesh=_sc_mesh(),
        out_type=[jax.ShapeDtypeStruct((EM + CH, LAT), _f32),
                  jax.ShapeDtypeStruct((EW + CH, LAT), _f32)],
        scratch_types=[
            pltpu.VMEM((npm, CH), jnp.int32),
            pltpu.VMEM((npm, CH), jnp.int32),
            pltpu.VMEM((npw, CH), jnp.int32),
            pltpu.VMEM((npw, CH), jnp.int32),
            [pltpu.VMEM((CH, LAT), _f32)] * 2,
            [pltpu.VMEM((CH, LAT), _f32)] * 2,
            [pltpu.SemaphoreType.DMA] * 2,
            [pltpu.SemaphoreType.DMA] * 2,
        ])
    def k(psm, prm, psw, prw, ism, irm, isw, irw, om, ow,
          ismv, irmv, iswv, irwv, rs, rr, sg, sw):
        cid = lax.axis_index("c")
        sid = lax.axis_index("s")
        wid = cid * NS + sid
        pltpu.sync_copy(ism.at[pl.ds(wid * npm, npm)], ismv)
        pltpu.sync_copy(irm.at[pl.ds(wid * npm, npm)], irmv)
        pltpu.sync_copy(isw.at[pl.ds(wid * npw, npw)], iswv)
        pltpu.sync_copy(irw.at[pl.ds(wid * npw, npw)], irwv)
        _gather_pipe(psm, prm, ismv, irmv, om, qm, rm, tcm, npm, wid,
                     rs, rr, sg, sw)
        _gather_pipe(psw, prw, iswv, irwv, ow, qw, rw, tcw, npw, wid,
                     rs, rr, sg, sw)

    return k


ZSTR = NP_ // NS


def _scatter_pipe(vals, iv, agg, q, r, tc_, nch_pad, wid, rv, sems):
    nch = q + jnp.where(wid < r, 1, 0)
    start = wid * q + jnp.minimum(wid, r)
    rvb = rv[0]

    def body(c, carry):
        pltpu.sync_copy(vals.at[pl.ds((start + c) * CH, CH)], rvb)
        pltpu.sync_copy(rvb, agg.at[iv.at[c]], add=True)
        return carry

    lax.fori_loop(0, nch, body, 0)


@functools.cache
def _make_scatter():
    tcm, qm, rm, _, npm = _chunking(EM, NW)
    tcw, qw, rw, _, npw = _chunking(EW, NW)
    sh = jax.ShapeDtypeStruct((NC * NP_, LAT), _f32)

    @functools.partial(
        pl.kernel, mesh=_sc_mesh(),
        out_type=[sh, sh],
        scratch_types=[
            pltpu.VMEM((npm, CH), jnp.int32),
            pltpu.VMEM((npw, CH), jnp.int32),
            [pltpu.VMEM((CH, LAT), _f32)] * 2,
            pltpu.VMEM_SHARED((NP_, LAT), _f32),
            [pltpu.SemaphoreType.DMA] * 2,
        ])
    def k(vm, vw, im, iw, zer, om, ow, imv, iwv, rv, agg, sems):
        cid = lax.axis_index("c")
        sid = lax.axis_index("s")
        wid = cid * NS + sid
        pltpu.sync_copy(im.at[pl.ds(wid * npm, npm)], imv)
        pltpu.sync_copy(iw.at[pl.ds(wid * npw, npw)], iwv)
        pltpu.sync_copy(zer.at[pl.ds(sid * ZSTR, ZSTR)],
                        agg.at[pl.ds(sid * ZSTR, ZSTR)])
        plsc.subcore_barrier()
        _scatter_pipe(vm, imv, agg, qm, rm, tcm, npm, wid, rv, sems)
        plsc.subcore_barrier()
        pltpu.sync_copy(agg.at[pl.ds(sid * ZSTR, ZSTR)],
                        om.at[pl.ds(cid * NP_ + sid * ZSTR, ZSTR)])
        pltpu.sync_copy(zer.at[pl.ds(sid * ZSTR, ZSTR)],
                        agg.at[pl.ds(sid * ZSTR, ZSTR)])
        plsc.subcore_barrier()
        _scatter_pipe(vw, iwv, agg, qw, rw, tcw, npw, wid, rv, sems)
        plsc.subcore_barrier()
        pltpu.sync_copy(agg.at[pl.ds(sid * ZSTR, ZSTR)],
                        ow.at[pl.ds(cid * NP_ + sid * ZSTR, ZSTR)])

    return k


def _slab_idx(idx, E, pad_val):
    tc, q, r, nch_max, nch_pad = _chunking(E, NW)
    perm = np.full((NW, nch_pad), tc, dtype=np.int32)
    for w in range(NW):
        s = w * q + min(w, r)
        n = q + (1 if w < r else 0)
        perm[w, :n] = np.arange(s, s + n, dtype=np.int32)
    idx2 = jnp.concatenate([idx.astype(jnp.int32).reshape(tc, CH),
                            jnp.full((1, CH), pad_val, jnp.int32)])
    return idx2[perm.reshape(-1)]


def kernel(node_features, mesh_features, world_features, params,
           mesh_senders, mesh_receivers, world_senders, world_receivers):
    enc = params["enc"]
    node_lat = _enc_call(node_features, enc["node"], 5000)
    mesh_lat = _enc_call(mesh_features, enc["mesh"], 6400)
    world_lat = _enc_call(world_features, enc["world"], 6400)

    ms2 = _slab_idx(mesh_senders, EM, 0)
    mr2 = _slab_idx(mesh_receivers, EM, 0)
    ws2 = _slab_idx(world_senders, EW, 0)
    wr2 = _slab_idx(world_receivers, EW, 0)
    smr = _slab_idx(mesh_receivers, EM, N)
    swr = _slab_idx(world_receivers, EW, N)
    zeros = jnp.zeros((NP_, LAT), _f32)

    gather = _make_gather()
    scatter = _make_scatter()

    for blk in params["blocks"]:
        pm, pw, pn = blk["mesh"], blk["world"], blk["node"]
        W1m, W1w = pm["W"][0], pw["W"][0]
        psm, prm, psw, prw = _proj_call(
            node_lat, W1m[:LAT], W1m[LAT:2 * LAT], pm["b"][0],
            W1w[:LAT], W1w[LAT:2 * LAT], pw["b"][0])
        Gm, Gw = gather(psm, prm, psw, prw, ms2, mr2, ws2, wr2)
        new_mesh, mesh_lat = _edge_call(Gm, mesh_lat, pm, W1m[2 * LAT:])
        new_world, world_lat = _edge_call(Gw, world_lat, pw, W1w[2 * LAT:])
        aggm, aggw = scatter(new_mesh, new_world, smr, swr, zeros)
        node_lat = _node_call(node_lat, aggm[:NP_], aggm[NP_:],
                              aggw[:NP_], aggw[NP_:], pn)

    return node_lat, mesh_lat, world_lat


# speedup vs baseline: 1.0720x; 1.0720x over previous
import functools

import numpy as np

import jax
import jax.numpy as jnp
from jax import lax
from jax.experimental import pallas as pl
from jax.experimental.pallas import tpu as pltpu
from jax.experimental.pallas import tpu_sc as plsc

N = 10000
EM = 160000
EW = 32000
LAT = 128
LN_EPS = 1e-5
NC = 2
NS = 16
NW = NC * NS
CH = 128
NP_ = 10240
SPT = NP_ // NS

_f32 = jnp.float32


def _rb(blk, d):
    return pl.BlockSpec((blk, d), lambda i: (i, 0))


def _full(shape):
    nd = len(shape)
    return pl.BlockSpec(shape, lambda i: (0,) * nd)


def _ln(h, g, b):
    mu = jnp.mean(h, axis=-1, keepdims=True)
    d = h - mu
    var = jnp.mean(d * d, axis=-1, keepdims=True)
    return d * lax.rsqrt(var + LN_EPS) * g + b


def _dot(a, b):
    return jnp.dot(a, b, preferred_element_type=_f32)


def _enc_body(x_ref, w1, w2, w3, w4, b1, b2, b3, b4, g, be, o_ref):
    h = jnp.maximum(_dot(x_ref[...], w1[...]) + b1[...], 0.0)
    h = jnp.maximum(_dot(h, w2[...]) + b2[...], 0.0)
    h = jnp.maximum(_dot(h, w3[...]) + b3[...], 0.0)
    h = _dot(h, w4[...]) + b4[...]
    o_ref[...] = _ln(h, g[...], be[...])


def _enc_call(x, p, blk):
    rows, din = x.shape
    Ws, bs = p["W"], p["b"]
    args = [x, Ws[0], Ws[1], Ws[2], Ws[3]] + [b.reshape(1, -1) for b in bs] \
        + [p["g"].reshape(1, -1), p["be"].reshape(1, -1)]
    specs = [_rb(blk, din)] + [_full(a.shape) for a in args[1:]]
    return pl.pallas_call(
        _enc_body,
        grid=(rows // blk,),
        in_specs=specs,
        out_specs=_rb(blk, LAT),
        out_shape=jax.ShapeDtypeStruct((rows, LAT), _f32),
    )(*args)


def _proj_body(x_ref, wam, wbm, b1m, waw, wbw, b1w, psm, prm, psw, prw):
    x = x_ref[...]
    psm[...] = _dot(x, wam[...])
    prm[...] = _dot(x, wbm[...]) + b1m[...]
    psw[...] = _dot(x, waw[...])
    prw[...] = _dot(x, wbw[...]) + b1w[...]


def _proj_call(nl, wam, wbm, b1m, waw, wbw, b1w, blk=5000):
    args = [nl, wam, wbm, b1m.reshape(1, -1), waw, wbw, b1w.reshape(1, -1)]
    specs = [_rb(blk, LAT)] + [_full(a.shape) for a in args[1:]]
    sh = jax.ShapeDtypeStruct((N, LAT), _f32)
    return pl.pallas_call(
        _proj_body,
        grid=(N // blk,),
        in_specs=specs,
        out_specs=[_rb(blk, LAT)] * 4,
        out_shape=[sh, sh, sh, sh],
    )(*args)


def _bdot(a, b):
    return jnp.dot(a.astype(jnp.bfloat16), b.astype(jnp.bfloat16),
                   preferred_element_type=_f32)


def _edge_body(g_ref, lat_ref, w1c, w2, w3, w4, b2, b3, b4, gm, be,
               new_ref, lat_o_ref):
    x = lat_ref[...]
    h = jnp.maximum(g_ref[...] + _bdot(x, w1c[...]), 0.0)
    h = jnp.maximum(_bdot(h, w2[...]) + b2[...], 0.0)
    h = jnp.maximum(_bdot(h, w3[...]) + b3[...], 0.0)
    h = _bdot(h, w4[...]) + b4[...]
    nm = _ln(h, gm[...], be[...])
    new_ref[...] = nm
    lat_o_ref[...] = x + nm


def _edge_call(G, lat, p, w1c, blk=6400):
    rows = lat.shape[0]
    Ws, bs = p["W"], p["b"]
    args = [G, lat, w1c, Ws[1], Ws[2], Ws[3],
            bs[1].reshape(1, -1), bs[2].reshape(1, -1), bs[3].reshape(1, -1),
            p["g"].reshape(1, -1), p["be"].reshape(1, -1)]
    specs = [_rb(blk, LAT), _rb(blk, LAT)] + [_full(a.shape) for a in args[2:]]
    sh = jax.ShapeDtypeStruct((rows, LAT), _f32)
    return pl.pallas_call(
        _edge_body,
        grid=(rows // blk,),
        in_specs=specs,
        out_specs=[_rb(blk, LAT), _rb(blk, LAT)],
        out_shape=[sh, sh],
    )(*args)


def _node_body(nl_ref, am0, am1, aw0, aw1, v1a, v1b, v1c, c1, v2, v3, v4,
               c2, c3, c4, gm, be, o_ref):
    x = nl_ref[...]
    h = (_dot(x, v1a[...]) + _dot(am0[...] + am1[...], v1b[...])
         + _dot(aw0[...] + aw1[...], v1c[...]) + c1[...])
    h = jnp.maximum(h, 0.0)
    h = jnp.maximum(_dot(h, v2[...]) + c2[...], 0.0)
    h = jnp.maximum(_dot(h, v3[...]) + c3[...], 0.0)
    h = _dot(h, v4[...]) + c4[...]
    o_ref[...] = x + _ln(h, gm[...], be[...])


def _node_call(nl, am0, am1, aw0, aw1, p, blk=5000):
    Ws, bs = p["W"], p["b"]
    V1 = Ws[0]
    args = [nl, am0, am1, aw0, aw1,
            V1[:LAT], V1[LAT:2 * LAT], V1[2 * LAT:], bs[0].reshape(1, -1),
            Ws[1], Ws[2], Ws[3],
            bs[1].reshape(1, -1), bs[2].reshape(1, -1), bs[3].reshape(1, -1),
            p["g"].reshape(1, -1), p["be"].reshape(1, -1)]
    specs = [_rb(blk, LAT)] * 5 + [_full(a.shape) for a in args[5:]]
    return pl.pallas_call(
        _node_body,
        grid=(N // blk,),
        in_specs=specs,
        out_specs=_rb(blk, LAT),
        out_shape=jax.ShapeDtypeStruct((N, LAT), _f32),
    )(*args)


def _chunking(E, nworkers):
    tc = E // CH
    q, r = divmod(tc, nworkers)
    nch_max = q + (1 if r else 0)
    nch_pad = -(-nch_max // 8) * 8
    return tc, q, r, nch_max, nch_pad

def _sc_mesh():
    return plsc.VectorSubcoreMesh(
        core_axis_name="c", subcore_axis_name="s", num_cores=NC,
        num_subcores=NS)


def _gather_pipe(ps, pr, isv, irv, out, q, r, tc_, nch_pad, wid,
                 rs, rr, sg, sw):
    nch = q + jnp.where(wid < r, 1, 0)
    start = wid * q + jnp.minimum(wid, r)
    rsb, rrb = rs[0], rr[0]

    def body(c, carry):
        d1 = pltpu.async_copy(ps.at[isv.at[c]], rsb, sg[0])
        d2 = pltpu.async_copy(pr.at[irv.at[c]], rrb, sg[1])
        d1.wait()
        d2.wait()

        def addb(rw, c2):
            for k8 in range(LAT // 16):
                s_ = pl.ds(k8 * 16, 16)
                rsb[rw, s_] = rsb[rw, s_] + rrb[rw, s_]
            return c2

        lax.fori_loop(0, CH, addb, 0)
        pltpu.sync_copy(rsb, out.at[pl.ds((start + c) * CH, CH)])
        return carry

    lax.fori_loop(0, nch, body, 0)


@functools.cache
def _make_gather(E):
    tc_, q, r, _, nch_pad = _chunking(E, NW)

    @functools.partial(
        pl.kernel, mesh=_sc_mesh(),
        out_type=jax.ShapeDtypeStruct((E + CH, LAT), _f32),
        scratch_types=[
            pltpu.VMEM((nch_pad, CH), jnp.int32),
            pltpu.VMEM((nch_pad, CH), jnp.int32),
            [pltpu.VMEM((CH, LAT), _f32)] * 2,
            [pltpu.VMEM((CH, LAT), _f32)] * 2,
            [pltpu.SemaphoreType.DMA] * 2,
            [pltpu.SemaphoreType.DMA] * 2,
        ])
    def k(ps, pr, is_, ir, out, isv, irv, rs, rr, sg, sw):
        cid = lax.axis_index("c")
        sid = lax.axis_index("s")
        wid = cid * NS + sid
        pltpu.sync_copy(is_.at[pl.ds(wid * nch_pad, nch_pad)], isv)
        pltpu.sync_copy(ir.at[pl.ds(wid * nch_pad, nch_pad)], irv)
        _gather_pipe(ps, pr, isv, irv, out, q, r, tc_, nch_pad, wid,
                     rs, rr, sg, sw)

    return k


ZSTR = NP_ // NS


def _scatter_pipe(vals, iv, agg, q, r, tc_, nch_pad, wid, rv, sems):
    nch = q + jnp.where(wid < r, 1, 0)
    start = wid * q + jnp.minimum(wid, r)
    rvb = rv[0]

    def body(c, carry):
        pltpu.sync_copy(vals.at[pl.ds((start + c) * CH, CH)], rvb)
        pltpu.sync_copy(rvb, agg.at[iv.at[c]], add=True)
        return carry

    lax.fori_loop(0, nch, body, 0)


@functools.cache
def _make_scatter(E):
    tc_, q, r, _, nch_pad = _chunking(E, NW)

    @functools.partial(
        pl.kernel, mesh=_sc_mesh(),
        out_type=jax.ShapeDtypeStruct((NC * NP_, LAT), _f32),
        scratch_types=[
            pltpu.VMEM((nch_pad, CH), jnp.int32),
            [pltpu.VMEM((CH, LAT), _f32)] * 2,
            pltpu.VMEM_SHARED((NP_, LAT), _f32),
            [pltpu.SemaphoreType.DMA] * 2,
        ])
    def k(vals, ir, zer, out, irv, rv, agg, sems):
        cid = lax.axis_index("c")
        sid = lax.axis_index("s")
        wid = cid * NS + sid
        pltpu.sync_copy(zer.at[pl.ds(sid * ZSTR, ZSTR)],
                        agg.at[pl.ds(sid * ZSTR, ZSTR)])
        pltpu.sync_copy(ir.at[pl.ds(wid * nch_pad, nch_pad)], irv)
        plsc.subcore_barrier()
        _scatter_pipe(vals, irv, agg, q, r, tc_, nch_pad, wid, rv, sems)
        plsc.subcore_barrier()
        pltpu.sync_copy(agg.at[pl.ds(sid * ZSTR, ZSTR)],
                        out.at[pl.ds(cid * NP_ + sid * ZSTR, ZSTR)])

    return k


def _slab_idx(idx, E, pad_val):
    tc, q, r, nch_max, nch_pad = _chunking(E, NW)
    perm = np.full((NW, nch_pad), tc, dtype=np.int32)
    for w in range(NW):
        s = w * q + min(w, r)
        n = q + (1 if w < r else 0)
        perm[w, :n] = np.arange(s, s + n, dtype=np.int32)
    idx2 = jnp.concatenate([idx.astype(jnp.int32).reshape(tc, CH),
                            jnp.full((1, CH), pad_val, jnp.int32)])
    return idx2[perm.reshape(-1)]


def kernel(node_features, mesh_features, world_features, params,
           mesh_senders, mesh_receivers, world_senders, world_receivers):
    enc = params["enc"]
    node_lat = _enc_call(node_features, enc["node"], 5000)
    mesh_lat = _enc_call(mesh_features, enc["mesh"], 6400)
    world_lat = _enc_call(world_features, enc["world"], 6400)

    ms2 = _slab_idx(mesh_senders, EM, 0)
    mr2 = _slab_idx(mesh_receivers, EM, 0)
    ws2 = _slab_idx(world_senders, EW, 0)
    wr2 = _slab_idx(world_receivers, EW, 0)
    zeros = jnp.zeros((NP_, LAT), _f32)

    gather_m = _make_gather(EM)
    gather_w = _make_gather(EW)
    scatter_m = _make_scatter(EM)
    scatter_w = _make_scatter(EW)

    for blk in params["blocks"]:
        pm, pw, pn = blk["mesh"], blk["world"], blk["node"]
        W1m, W1w = pm["W"][0], pw["W"][0]
        psm, prm, psw, prw = _proj_call(
            node_lat, W1m[:LAT], W1m[LAT:2 * LAT], pm["b"][0],
            W1w[:LAT], W1w[LAT:2 * LAT], pw["b"][0])
        Gm = gather_m(psm, prm, ms2, mr2)
        Gw = gather_w(psw, prw, ws2, wr2)
        new_mesh, mesh_lat = _edge_call(Gm, mesh_lat, pm, W1m[2 * LAT:])
        new_world, world_lat = _edge_call(Gw, world_lat, pw, W1w[2 * LAT:])
        aggm = scatter_m(new_mesh, mr2, zeros)
        aggw = scatter_w(new_world, wr2, zeros)
        node_lat = _node_call(node_lat, aggm[:NP_], aggm[NP_:],
                              aggw[:NP_], aggw[NP_:], pn)

    return node_lat, mesh_lat, world_lat


# speedup vs baseline: 1.0790x; 1.0066x over previous
import functools

import numpy as np

import jax
import jax.numpy as jnp
from jax import lax
from jax.experimental import pallas as pl
from jax.experimental.pallas import tpu as pltpu
from jax.experimental.pallas import tpu_sc as plsc

N = 10000
EM = 160000
EW = 32000
LAT = 128
LN_EPS = 1e-5
NC = 2
NS = 16
NW = NC * NS
CH = 128
NP_ = 10240
SPT = NP_ // NS

_f32 = jnp.float32


def _rb(blk, d):
    return pl.BlockSpec((blk, d), lambda i: (i, 0))


def _full(shape):
    nd = len(shape)
    return pl.BlockSpec(shape, lambda i: (0,) * nd)


def _ln(h, g, b):
    mu = jnp.mean(h, axis=-1, keepdims=True)
    d = h - mu
    var = jnp.mean(d * d, axis=-1, keepdims=True)
    return d * lax.rsqrt(var + LN_EPS) * g + b


def _dot(a, b):
    return jnp.dot(a, b, preferred_element_type=_f32)


def _enc_body(x_ref, w1, w2, w3, w4, b1, b2, b3, b4, g, be, o_ref):
    h = jnp.maximum(_dot(x_ref[...], w1[...]) + b1[...], 0.0)
    h = jnp.maximum(_dot(h, w2[...]) + b2[...], 0.0)
    h = jnp.maximum(_dot(h, w3[...]) + b3[...], 0.0)
    h = _dot(h, w4[...]) + b4[...]
    o_ref[...] = _ln(h, g[...], be[...])


def _enc_call(x, p, blk):
    rows, din = x.shape
    Ws, bs = p["W"], p["b"]
    args = [x, Ws[0], Ws[1], Ws[2], Ws[3]] + [b.reshape(1, -1) for b in bs] \
        + [p["g"].reshape(1, -1), p["be"].reshape(1, -1)]
    specs = [_rb(blk, din)] + [_full(a.shape) for a in args[1:]]
    return pl.pallas_call(
        _enc_body,
        grid=(rows // blk,),
        in_specs=specs,
        out_specs=_rb(blk, LAT),
        out_shape=jax.ShapeDtypeStruct((rows, LAT), _f32),
    )(*args)


def _proj_body(x_ref, wam, wbm, b1m, waw, wbw, b1w, psm, prm, psw, prw):
    x = x_ref[...]
    psm[...] = _dot(x, wam[...])
    prm[...] = _dot(x, wbm[...]) + b1m[...]
    psw[...] = _dot(x, waw[...])
    prw[...] = _dot(x, wbw[...]) + b1w[...]


def _proj_call(nl, wam, wbm, b1m, waw, wbw, b1w, blk=5000):
    args = [nl, wam, wbm, b1m.reshape(1, -1), waw, wbw, b1w.reshape(1, -1)]
    specs = [_rb(blk, LAT)] + [_full(a.shape) for a in args[1:]]
    sh = jax.ShapeDtypeStruct((N, LAT), _f32)
    return pl.pallas_call(
        _proj_body,
        grid=(N // blk,),
        in_specs=specs,
        out_specs=[_rb(blk, LAT)] * 4,
        out_shape=[sh, sh, sh, sh],
    )(*args)


def _bdot(a, b):
    return jnp.dot(a.astype(jnp.bfloat16), b.astype(jnp.bfloat16),
                   preferred_element_type=_f32)


def _edge_body(g_ref, lat_ref, w1c, w2, w3, w4, b2, b3, b4, gm, be,
               new_ref, lat_o_ref):
    x = lat_ref[...]
    h = jnp.maximum(g_ref[...] + _bdot(x, w1c[...]), 0.0)
    h = jnp.maximum(_bdot(h, w2[...]) + b2[...], 0.0)
    h = jnp.maximum(_bdot(h, w3[...]) + b3[...], 0.0)
    h = _bdot(h, w4[...]) + b4[...]
    nm = _ln(h, gm[...], be[...])
    new_ref[...] = nm
    lat_o_ref[...] = x + nm


def _edge_call(G, lat, p, w1c, blk=6400):
    rows = lat.shape[0]
    Ws, bs = p["W"], p["b"]
    args = [G, lat, w1c, Ws[1], Ws[2], Ws[3],
            bs[1].reshape(1, -1), bs[2].reshape(1, -1), bs[3].reshape(1, -1),
            p["g"].reshape(1, -1), p["be"].reshape(1, -1)]
    specs = [_rb(blk, LAT), _rb(blk, LAT)] + [_full(a.shape) for a in args[2:]]
    sh = jax.ShapeDtypeStruct((rows, LAT), _f32)
    return pl.pallas_call(
        _edge_body,
        grid=(rows // blk,),
        in_specs=specs,
        out_specs=[_rb(blk, LAT), _rb(blk, LAT)],
        out_shape=[sh, sh],
    )(*args)


def _node_body(nl_ref, am0, am1, aw0, aw1, v1a, v1b, v1c, c1, v2, v3, v4,
               c2, c3, c4, gm, be, o_ref):
    x = nl_ref[...]
    h = (_dot(x, v1a[...]) + _dot(am0[...] + am1[...], v1b[...])
         + _dot(aw0[...] + aw1[...], v1c[...]) + c1[...])
    h = jnp.maximum(h, 0.0)
    h = jnp.maximum(_dot(h, v2[...]) + c2[...], 0.0)
    h = jnp.maximum(_dot(h, v3[...]) + c3[...], 0.0)
    h = _dot(h, v4[...]) + c4[...]
    o_ref[...] = x + _ln(h, gm[...], be[...])


def _node_proj_body(nl_ref, am0, am1, aw0, aw1, v1a, v1b, v1c, c1,
                    v2, v3, v4, c2, c3, c4, gm, be,
                    wam, wbm, b1m, waw, wbw, b1w,
                    o_ref, psm, prm, psw, prw):
    x = nl_ref[...]
    h = (_dot(x, v1a[...]) + _dot(am0[...] + am1[...], v1b[...])
         + _dot(aw0[...] + aw1[...], v1c[...]) + c1[...])
    h = jnp.maximum(h, 0.0)
    h = jnp.maximum(_dot(h, v2[...]) + c2[...], 0.0)
    h = jnp.maximum(_dot(h, v3[...]) + c3[...], 0.0)
    h = _dot(h, v4[...]) + c4[...]
    o = x + _ln(h, gm[...], be[...])
    o_ref[...] = o
    psm[...] = _dot(o, wam[...])
    prm[...] = _dot(o, wbm[...]) + b1m[...]
    psw[...] = _dot(o, waw[...])
    prw[...] = _dot(o, wbw[...]) + b1w[...]


def _node_proj_call(nl, am0, am1, aw0, aw1, p, nxt_m, nxt_w, blk=5000):
    Ws, bs = p["W"], p["b"]
    V1 = Ws[0]
    W1m, W1w = nxt_m["W"][0], nxt_w["W"][0]
    args = [nl, am0, am1, aw0, aw1,
            V1[:LAT], V1[LAT:2 * LAT], V1[2 * LAT:], bs[0].reshape(1, -1),
            Ws[1], Ws[2], Ws[3],
            bs[1].reshape(1, -1), bs[2].reshape(1, -1), bs[3].reshape(1, -1),
            p["g"].reshape(1, -1), p["be"].reshape(1, -1),
            W1m[:LAT], W1m[LAT:2 * LAT], nxt_m["b"][0].reshape(1, -1),
            W1w[:LAT], W1w[LAT:2 * LAT], nxt_w["b"][0].reshape(1, -1)]
    specs = [_rb(blk, LAT)] * 5 + [_full(a.shape) for a in args[5:]]
    sh = jax.ShapeDtypeStruct((N, LAT), _f32)
    return pl.pallas_call(
        _node_proj_body,
        grid=(N // blk,),
        in_specs=specs,
        out_specs=[_rb(blk, LAT)] * 5,
        out_shape=[sh, sh, sh, sh, sh],
    )(*args)


def _node_call(nl, am0, am1, aw0, aw1, p, blk=5000):
    Ws, bs = p["W"], p["b"]
    V1 = Ws[0]
    args = [nl, am0, am1, aw0, aw1,
            V1[:LAT], V1[LAT:2 * LAT], V1[2 * LAT:], bs[0].reshape(1, -1),
            Ws[1], Ws[2], Ws[3],
            bs[1].reshape(1, -1), bs[2].reshape(1, -1), bs[3].reshape(1, -1),
            p["g"].reshape(1, -1), p["be"].reshape(1, -1)]
    specs = [_rb(blk, LAT)] * 5 + [_full(a.shape) for a in args[5:]]
    return pl.pallas_call(
        _node_body,
        grid=(N // blk,),
        in_specs=specs,
        out_specs=_rb(blk, LAT),
        out_shape=jax.ShapeDtypeStruct((N, LAT), _f32),
    )(*args)


def _chunking(E, nworkers):
    tc = E // CH
    q, r = divmod(tc, nworkers)
    nch_max = q + (1 if r else 0)
    nch_pad = -(-nch_max // 8) * 8
    return tc, q, r, nch_max, nch_pad

def _sc_mesh():
    return plsc.VectorSubcoreMesh(
        core_axis_name="c", subcore_axis_name="s", num_cores=NC,
        num_subcores=NS)


def _gather_pipe(ps, pr, isv, irv, out, q, r, tc_, nch_pad, wid,
                 rs, rr, sg, sw):
    nch = q + jnp.where(wid < r, 1, 0)
    start = wid * q + jnp.minimum(wid, r)
    rsb, rrb = rs[0], rr[0]

    def body(c, carry):
        d1 = pltpu.async_copy(ps.at[isv.at[c]], rsb, sg[0])
        d2 = pltpu.async_copy(pr.at[irv.at[c]], rrb, sg[1])
        d1.wait()
        d2.wait()

        def addb(rw, c2):
            for k8 in range(LAT // 16):
                s_ = pl.ds(k8 * 16, 16)
                rsb[rw, s_] = rsb[rw, s_] + rrb[rw, s_]
            return c2

        lax.fori_loop(0, CH, addb, 0)
        pltpu.sync_copy(rsb, out.at[pl.ds((start + c) * CH, CH)])
        return carry

    lax.fori_loop(0, nch, body, 0)


@functools.cache
def _make_gather(E):
    tc_, q, r, _, nch_pad = _chunking(E, NW)

    @functools.partial(
        pl.kernel, mesh=_sc_mesh(),
        out_type=jax.ShapeDtypeStruct((E + CH, LAT), _f32),
        scratch_types=[
            pltpu.VMEM((nch_pad, CH), jnp.int32),
            pltpu.VMEM((nch_pad, CH), jnp.int32),
            [pltpu.VMEM((CH, LAT), _f32)] * 2,
            [pltpu.VMEM((CH, LAT), _f32)] * 2,
            [pltpu.SemaphoreType.DMA] * 2,
            [pltpu.SemaphoreType.DMA] * 2,
        ])
    def k(ps, pr, is_, ir, out, isv, irv, rs, rr, sg, sw):
        cid = lax.axis_index("c")
        sid = lax.axis_index("s")
        wid = cid * NS + sid
        pltpu.sync_copy(is_.at[pl.ds(wid * nch_pad, nch_pad)], isv)
        pltpu.sync_copy(ir.at[pl.ds(wid * nch_pad, nch_pad)], irv)
        _gather_pipe(ps, pr, isv, irv, out, q, r, tc_, nch_pad, wid,
                     rs, rr, sg, sw)

    return k


ZSTR = NP_ // NS


def _scatter_pipe(vals, iv, agg, q, r, tc_, nch_pad, wid, rv, sems):
    nch = q + jnp.where(wid < r, 1, 0)
    start = wid * q + jnp.minimum(wid, r)
    rvb = rv[0]

    def body(c, carry):
        pltpu.sync_copy(vals.at[pl.ds((start + c) * CH, CH)], rvb)
        pltpu.sync_copy(rvb, agg.at[iv.at[c]], add=True)
        return carry

    lax.fori_loop(0, nch, body, 0)


@functools.cache
def _make_scatter(E):
    tc_, q, r, _, nch_pad = _chunking(E, NW)

    @functools.partial(
        pl.kernel, mesh=_sc_mesh(),
        out_type=jax.ShapeDtypeStruct((NC * NP_, LAT), _f32),
        scratch_types=[
            pltpu.VMEM((nch_pad, CH), jnp.int32),
            [pltpu.VMEM((CH, LAT), _f32)] * 2,
            pltpu.VMEM_SHARED((NP_, LAT), _f32),
            [pltpu.SemaphoreType.DMA] * 2,
        ])
    def k(vals, ir, zer, out, irv, rv, agg, sems):
        cid = lax.axis_index("c")
        sid = lax.axis_index("s")
        wid = cid * NS + sid
        pltpu.sync_copy(zer.at[pl.ds(sid * ZSTR, ZSTR)],
                        agg.at[pl.ds(sid * ZSTR, ZSTR)])
        pltpu.sync_copy(ir.at[pl.ds(wid * nch_pad, nch_pad)], irv)
        plsc.subcore_barrier()
        _scatter_pipe(vals, irv, agg, q, r, tc_, nch_pad, wid, rv, sems)
        plsc.subcore_barrier()
        pltpu.sync_copy(agg.at[pl.ds(sid * ZSTR, ZSTR)],
                        out.at[pl.ds(cid * NP_ + sid * ZSTR, ZSTR)])

    return k


def _slab_idx(idx, E, pad_val):
    tc, q, r, nch_max, nch_pad = _chunking(E, NW)
    perm = np.full((NW, nch_pad), tc, dtype=np.int32)
    for w in range(NW):
        s = w * q + min(w, r)
        n = q + (1 if w < r else 0)
        perm[w, :n] = np.arange(s, s + n, dtype=np.int32)
    idx2 = jnp.concatenate([idx.astype(jnp.int32).reshape(tc, CH),
                            jnp.full((1, CH), pad_val, jnp.int32)])
    return idx2[perm.reshape(-1)]


def kernel(node_features, mesh_features, world_features, params,
           mesh_senders, mesh_receivers, world_senders, world_receivers):
    enc = params["enc"]
    node_lat = _enc_call(node_features, enc["node"], 5000)
    mesh_lat = _enc_call(mesh_features, enc["mesh"], 6400)
    world_lat = _enc_call(world_features, enc["world"], 6400)

    ms2 = _slab_idx(mesh_senders, EM, 0)
    mr2 = _slab_idx(mesh_receivers, EM, 0)
    ws2 = _slab_idx(world_senders, EW, 0)
    wr2 = _slab_idx(world_receivers, EW, 0)
    zeros = jnp.zeros((NP_, LAT), _f32)

    gather_m = _make_gather(EM)
    gather_w = _make_gather(EW)
    scatter_m = _make_scatter(EM)
    scatter_w = _make_scatter(EW)

    blocks = params["blocks"]
    b0m, b0w = blocks[0]["mesh"], blocks[0]["world"]
    projs = _proj_call(
        node_lat, b0m["W"][0][:LAT], b0m["W"][0][LAT:2 * LAT], b0m["b"][0],
        b0w["W"][0][:LAT], b0w["W"][0][LAT:2 * LAT], b0w["b"][0])

    for s, blk in enumerate(blocks):
        pm, pw, pn = blk["mesh"], blk["world"], blk["node"]
        W1m, W1w = pm["W"][0], pw["W"][0]
        psm, prm, psw, prw = projs
        Gm = gather_m(psm, prm, ms2, mr2)
        Gw = gather_w(psw, prw, ws2, wr2)
        new_mesh, mesh_lat = _edge_call(Gm, mesh_lat, pm, W1m[2 * LAT:])
        new_world, world_lat = _edge_call(Gw, world_lat, pw, W1w[2 * LAT:])
        aggm = scatter_m(new_mesh, mr2, zeros)
        aggw = scatter_w(new_world, wr2, zeros)
        if s + 1 < len(blocks):
            node_lat, *projs = _node_proj_call(
                node_lat, aggm[:NP_], aggm[NP_:], aggw[:NP_], aggw[NP_:],
                pn, blocks[s + 1]["mesh"], blocks[s + 1]["world"])
        else:
            node_lat = _node_call(node_lat, aggm[:NP_], aggm[NP_:],
                                  aggw[:NP_], aggw[NP_:], pn)

    return node_lat, mesh_lat, world_lat


# speedup vs baseline: 1.0803x; 1.0012x over previous
import functools

import numpy as np

import jax
import jax.numpy as jnp
from jax import lax
from jax.experimental import pallas as pl
from jax.experimental.pallas import tpu as pltpu
from jax.experimental.pallas import tpu_sc as plsc

N = 10000
EM = 160000
EW = 32000
LAT = 128
LN_EPS = 1e-5
NC = 2
NS = 16
NW = NC * NS
CH = 128
NP_ = 10240

_f32 = jnp.float32


def _rb(blk, d):
    return pl.BlockSpec((blk, d), lambda i: (i, 0))


def _full(shape):
    nd = len(shape)
    return pl.BlockSpec(shape, lambda i: (0,) * nd)


def _ln(h, g, b):
    mu = jnp.mean(h, axis=-1, keepdims=True)
    d = h - mu
    var = jnp.mean(d * d, axis=-1, keepdims=True)
    return d * lax.rsqrt(var + LN_EPS) * g + b


def _dot(a, b):
    return jnp.dot(a, b, preferred_element_type=_f32)


def _enc_body(x_ref, w1, w2, w3, w4, b1, b2, b3, b4, g, be, o_ref):
    h = jnp.maximum(_dot(x_ref[...], w1[...]) + b1[...], 0.0)
    h = jnp.maximum(_dot(h, w2[...]) + b2[...], 0.0)
    h = jnp.maximum(_dot(h, w3[...]) + b3[...], 0.0)
    h = _dot(h, w4[...]) + b4[...]
    o_ref[...] = _ln(h, g[...], be[...])


def _enc_call(x, p, blk):
    rows, din = x.shape
    Ws, bs = p["W"], p["b"]
    args = [x, Ws[0], Ws[1], Ws[2], Ws[3]] + [b.reshape(1, -1) for b in bs] \
        + [p["g"].reshape(1, -1), p["be"].reshape(1, -1)]
    specs = [_rb(blk, din)] + [_full(a.shape) for a in args[1:]]
    return pl.pallas_call(
        _enc_body,
        grid=(rows // blk,),
        in_specs=specs,
        out_specs=_rb(blk, LAT),
        out_shape=jax.ShapeDtypeStruct((rows, LAT), _f32),
    )(*args)


def _proj_body(x_ref, wam, wbm, b1m, waw, wbw, b1w, psm, prm, psw, prw):
    x = x_ref[...]
    psm[...] = _dot(x, wam[...])
    prm[...] = _dot(x, wbm[...]) + b1m[...]
    psw[...] = _dot(x, waw[...])
    prw[...] = _dot(x, wbw[...]) + b1w[...]


def _proj_call(nl, wam, wbm, b1m, waw, wbw, b1w, blk=5000):
    args = [nl, wam, wbm, b1m.reshape(1, -1), waw, wbw, b1w.reshape(1, -1)]
    specs = [_rb(blk, LAT)] + [_full(a.shape) for a in args[1:]]
    sh = jax.ShapeDtypeStruct((N, LAT), _f32)
    return pl.pallas_call(
        _proj_body,
        grid=(N // blk,),
        in_specs=specs,
        out_specs=[_rb(blk, LAT)] * 4,
        out_shape=[sh, sh, sh, sh],
    )(*args)


def _bdot(a, b):
    return jnp.dot(a.astype(jnp.bfloat16), b.astype(jnp.bfloat16),
                   preferred_element_type=_f32)


def _edge_body(g_ref, lat_ref, w1c, w2, w3, w4, b2, b3, b4, gm, be,
               new_ref, lat_o_ref):
    x = lat_ref[...]
    h = jnp.maximum(g_ref[...] + _bdot(x, w1c[...]), 0.0)
    h = jnp.maximum(_bdot(h, w2[...]) + b2[...], 0.0)
    h = jnp.maximum(_bdot(h, w3[...]) + b3[...], 0.0)
    h = _bdot(h, w4[...]) + b4[...]
    nm = _ln(h, gm[...], be[...])
    new_ref[...] = nm
    lat_o_ref[...] = x + nm


def _edge_call(G, lat, p, w1c, blk=6400):
    rows = lat.shape[0]
    Ws, bs = p["W"], p["b"]
    args = [G, lat, w1c, Ws[1], Ws[2], Ws[3],
            bs[1].reshape(1, -1), bs[2].reshape(1, -1), bs[3].reshape(1, -1),
            p["g"].reshape(1, -1), p["be"].reshape(1, -1)]
    specs = [_rb(blk, LAT), _rb(blk, LAT)] + [_full(a.shape) for a in args[2:]]
    sh = jax.ShapeDtypeStruct((rows, LAT), _f32)
    return pl.pallas_call(
        _edge_body,
        grid=(rows // blk,),
        in_specs=specs,
        out_specs=[_rb(blk, LAT), _rb(blk, LAT)],
        out_shape=[sh, sh],
    )(*args)


def _node_body(nl_ref, am0, am1, aw0, aw1, v1a, v1b, v1c, c1, v2, v3, v4,
               c2, c3, c4, gm, be, o_ref):
    x = nl_ref[...]
    h = (_dot(x, v1a[...]) + _dot(am0[...] + am1[...], v1b[...])
         + _dot(aw0[...] + aw1[...], v1c[...]) + c1[...])
    h = jnp.maximum(h, 0.0)
    h = jnp.maximum(_dot(h, v2[...]) + c2[...], 0.0)
    h = jnp.maximum(_dot(h, v3[...]) + c3[...], 0.0)
    h = _dot(h, v4[...]) + c4[...]
    o_ref[...] = x + _ln(h, gm[...], be[...])


def _node_proj_body(nl_ref, am0, am1, aw0, aw1, v1a, v1b, v1c, c1,
                    v2, v3, v4, c2, c3, c4, gm, be,
                    wam, wbm, b1m, waw, wbw, b1w,
                    o_ref, psm, prm, psw, prw):
    x = nl_ref[...]
    h = (_dot(x, v1a[...]) + _dot(am0[...] + am1[...], v1b[...])
         + _dot(aw0[...] + aw1[...], v1c[...]) + c1[...])
    h = jnp.maximum(h, 0.0)
    h = jnp.maximum(_dot(h, v2[...]) + c2[...], 0.0)
    h = jnp.maximum(_dot(h, v3[...]) + c3[...], 0.0)
    h = _dot(h, v4[...]) + c4[...]
    o = x + _ln(h, gm[...], be[...])
    o_ref[...] = o
    psm[...] = _dot(o, wam[...])
    prm[...] = _dot(o, wbm[...]) + b1m[...]
    psw[...] = _dot(o, waw[...])
    prw[...] = _dot(o, wbw[...]) + b1w[...]


def _node_proj_call(nl, am0, am1, aw0, aw1, p, nxt_m, nxt_w, blk=5000):
    Ws, bs = p["W"], p["b"]
    V1 = Ws[0]
    W1m, W1w = nxt_m["W"][0], nxt_w["W"][0]
    args = [nl, am0, am1, aw0, aw1,
            V1[:LAT], V1[LAT:2 * LAT], V1[2 * LAT:], bs[0].reshape(1, -1),
            Ws[1], Ws[2], Ws[3],
            bs[1].reshape(1, -1), bs[2].reshape(1, -1), bs[3].reshape(1, -1),
            p["g"].reshape(1, -1), p["be"].reshape(1, -1),
            W1m[:LAT], W1m[LAT:2 * LAT], nxt_m["b"][0].reshape(1, -1),
            W1w[:LAT], W1w[LAT:2 * LAT], nxt_w["b"][0].reshape(1, -1)]
    specs = [_rb(blk, LAT)] * 5 + [_full(a.shape) for a in args[5:]]
    sh = jax.ShapeDtypeStruct((N, LAT), _f32)
    return pl.pallas_call(
        _node_proj_body,
        grid=(N // blk,),
        in_specs=specs,
        out_specs=[_rb(blk, LAT)] * 5,
        out_shape=[sh, sh, sh, sh, sh],
    )(*args)


def _node_call(nl, am0, am1, aw0, aw1, p, blk=5000):
    Ws, bs = p["W"], p["b"]
    V1 = Ws[0]
    args = [nl, am0, am1, aw0, aw1,
            V1[:LAT], V1[LAT:2 * LAT], V1[2 * LAT:], bs[0].reshape(1, -1),
            Ws[1], Ws[2], Ws[3],
            bs[1].reshape(1, -1), bs[2].reshape(1, -1), bs[3].reshape(1, -1),
            p["g"].reshape(1, -1), p["be"].reshape(1, -1)]
    specs = [_rb(blk, LAT)] * 5 + [_full(a.shape) for a in args[5:]]
    return pl.pallas_call(
        _node_body,
        grid=(N // blk,),
        in_specs=specs,
        out_specs=_rb(blk, LAT),
        out_shape=jax.ShapeDtypeStruct((N, LAT), _f32),
    )(*args)


def _chunking(E, nworkers):
    tc = E // CH
    q, r = divmod(tc, nworkers)
    nch_max = q + (1 if r else 0)
    nch_pad = -(-nch_max // 8) * 8
    return tc, q, r, nch_max, nch_pad

def _sc_mesh():
    return plsc.VectorSubcoreMesh(
        core_axis_name="c", subcore_axis_name="s", num_cores=NC,
        num_subcores=NS)


def _gather_pipe(ps, pr, isv, irv, out, q, r, tc_, nch_pad, wid,
                 rs, rr, sg, sw):
    nch = q + jnp.where(wid < r, 1, 0)
    start = wid * q + jnp.minimum(wid, r)
    rsb, rrb = rs[0], rr[0]

    def body(c, carry):
        d1 = pltpu.async_copy(ps.at[isv.at[c]], rsb, sg[0])
        d2 = pltpu.async_copy(pr.at[irv.at[c]], rrb, sg[1])
        d1.wait()
        d2.wait()

        def addb(rw, c2):
            for k8 in range(LAT // 16):
                s_ = pl.ds(k8 * 16, 16)
                rsb[rw, s_] = rsb[rw, s_] + rrb[rw, s_]
            return c2

        lax.fori_loop(0, CH, addb, 0)
        pltpu.sync_copy(rsb, out.at[pl.ds((start + c) * CH, CH)])
        return carry

    lax.fori_loop(0, nch, body, 0)


@functools.cache
def _make_gather(E):
    tc_, q, r, _, nch_pad = _chunking(E, NW)

    @functools.partial(
        pl.kernel, mesh=_sc_mesh(),
        out_type=jax.ShapeDtypeStruct((E + CH, LAT), _f32),
        scratch_types=[
            pltpu.VMEM((nch_pad, CH), jnp.int32),
            pltpu.VMEM((nch_pad, CH), jnp.int32),
            [pltpu.VMEM((CH, LAT), _f32)] * 2,
            [pltpu.VMEM((CH, LAT), _f32)] * 2,
            [pltpu.SemaphoreType.DMA] * 2,
            [pltpu.SemaphoreType.DMA] * 2,
        ])
    def k(ps, pr, is_, ir, out, isv, irv, rs, rr, sg, sw):
        cid = lax.axis_index("c")
        sid = lax.axis_index("s")
        wid = cid * NS + sid
        pltpu.sync_copy(is_.at[pl.ds(wid * nch_pad, nch_pad)], isv)
        pltpu.sync_copy(ir.at[pl.ds(wid * nch_pad, nch_pad)], irv)
        _gather_pipe(ps, pr, isv, irv, out, q, r, tc_, nch_pad, wid,
                     rs, rr, sg, sw)

    return k


ZSTR = NP_ // NS


def _scatter_pipe(vals, iv, agg, q, r, tc_, nch_pad, wid, rv, sems):
    nch = q + jnp.where(wid < r, 1, 0)
    start = wid * q + jnp.minimum(wid, r)
    rvb = rv[0]

    def body(c, carry):
        pltpu.sync_copy(vals.at[pl.ds((start + c) * CH, CH)], rvb)
        pltpu.sync_copy(rvb, agg.at[iv.at[c]], add=True)
        return carry

    lax.fori_loop(0, nch, body, 0)


@functools.cache
def _make_scatter(E):
    tc_, q, r, _, nch_pad = _chunking(E, NW)

    @functools.partial(
        pl.kernel, mesh=_sc_mesh(),
        out_type=jax.ShapeDtypeStruct((NC * NP_, LAT), _f32),
        scratch_types=[
            pltpu.VMEM((nch_pad, CH), jnp.int32),
            [pltpu.VMEM((CH, LAT), _f32)] * 2,
            pltpu.VMEM_SHARED((NP_, LAT), _f32),
            [pltpu.SemaphoreType.DMA] * 2,
        ])
    def k(vals, ir, zer, out, irv, rv, agg, sems):
        cid = lax.axis_index("c")
        sid = lax.axis_index("s")
        wid = cid * NS + sid
        pltpu.sync_copy(zer.at[pl.ds(sid * ZSTR, ZSTR)],
                        agg.at[pl.ds(sid * ZSTR, ZSTR)])
        pltpu.sync_copy(ir.at[pl.ds(wid * nch_pad, nch_pad)], irv)
        plsc.subcore_barrier()
        _scatter_pipe(vals, irv, agg, q, r, tc_, nch_pad, wid, rv, sems)
        plsc.subcore_barrier()
        pltpu.sync_copy(agg.at[pl.ds(sid * ZSTR, ZSTR)],
                        out.at[pl.ds(cid * NP_ + sid * ZSTR, ZSTR)])

    return k


def _slab_idx(idx, E, pad_val):
    tc, q, r, nch_max, nch_pad = _chunking(E, NW)
    perm = np.full((NW, nch_pad), tc, dtype=np.int32)
    for w in range(NW):
        s = w * q + min(w, r)
        n = q + (1 if w < r else 0)
        perm[w, :n] = np.arange(s, s + n, dtype=np.int32)
    idx2 = jnp.concatenate([idx.astype(jnp.int32).reshape(tc, CH),
                            jnp.full((1, CH), pad_val, jnp.int32)])
    return idx2[perm.reshape(-1)]


def kernel(node_features, mesh_features, world_features, params,
           mesh_senders, mesh_receivers, world_senders, world_receivers):
    enc = params["enc"]
    node_lat = _enc_call(node_features, enc["node"], 5000)
    mesh_lat = _enc_call(mesh_features, enc["mesh"], 6400)
    world_lat = _enc_call(world_features, enc["world"], 6400)

    ms2 = _slab_idx(mesh_senders, EM, 0)
    mr2 = _slab_idx(mesh_receivers, EM, 0)
    ws2 = _slab_idx(world_senders, EW, 0)
    wr2 = _slab_idx(world_receivers, EW, 0)
    zeros = jnp.zeros((NP_, LAT), _f32)

    gather_m = _make_gather(EM)
    gather_w = _make_gather(EW)
    scatter_m = _make_scatter(EM)
    scatter_w = _make_scatter(EW)

    blocks = params["blocks"]
    b0m, b0w = blocks[0]["mesh"], blocks[0]["world"]
    projs = _proj_call(
        node_lat, b0m["W"][0][:LAT], b0m["W"][0][LAT:2 * LAT], b0m["b"][0],
        b0w["W"][0][:LAT], b0w["W"][0][LAT:2 * LAT], b0w["b"][0])

    for s, blk in enumerate(blocks):
        pm, pw, pn = blk["mesh"], blk["world"], blk["node"]
        W1m, W1w = pm["W"][0], pw["W"][0]
        psm, prm, psw, prw = projs
        Gm = gather_m(psm, prm, ms2, mr2)
        Gw = gather_w(psw, prw, ws2, wr2)
        new_mesh, mesh_lat = _edge_call(Gm, mesh_lat, pm, W1m[2 * LAT:])
        new_world, world_lat = _edge_call(Gw, world_lat, pw, W1w[2 * LAT:])
        aggm = scatter_m(new_mesh, mr2, zeros)
        aggw = scatter_w(new_world, wr2, zeros)
        if s + 1 < len(blocks):
            node_lat, *projs = _node_proj_call(
                node_lat, aggm[:NP_], aggm[NP_:], aggw[:NP_], aggw[NP_:],
                pn, blocks[s + 1]["mesh"], blocks[s + 1]["world"])
        else:
            node_lat = _node_call(node_lat, aggm[:NP_], aggm[NP_:],
                                  aggw[:NP_], aggw[NP_:], pn)

    return node_lat, mesh_lat, world_lat
